# baseline (speedup 1.0000x reference)
import functools

import jax
import jax.numpy as jnp
from jax import lax
from jax.experimental import pallas as pl
from jax.experimental.pallas import tpu as pltpu
from jax.experimental.pallas import tpu_sc as plsc

N, E, D_IN, D_H, D_OUT = 10000, 320000, 128, 128, 64
NC, NS = 2, 16
NW = NC * NS
N_PAD = 10112
RPT = N_PAD // NS
C = 200
DEG_W = 8
DF = 64


def _mesh():
    return plsc.VectorSubcoreMesh(core_axis_name="c", subcore_axis_name="s")


_SC_PARAMS = pltpu.CompilerParams(use_tc_tiling_on_sc=False)


def _sc_degree(dst, ones_c, zeros16):
    ept = E // NW

    @functools.partial(
        pl.kernel,
        out_type=jax.ShapeDtypeStruct((NC, N_PAD, DEG_W), jnp.float32),
        mesh=_mesh(),
        scratch_types=[
            pltpu.VMEM((C,), jnp.int32),
            pltpu.VMEM((C,), jnp.int32),
            pltpu.VMEM((C, DEG_W), jnp.float32),
            pltpu.VMEM_SHARED((N_PAD, DEG_W), jnp.float32),
            pltpu.SemaphoreType.DMA,
        ],
        compiler_params=_SC_PARAMS,
    )
    def k(dst_h, ones_h, zeros_h, out_h, idx_a, idx_b, ones_v, acc, sem):
        c = lax.axis_index("c")
        s = lax.axis_index("s")
        gwid = c * NS + s
        r0 = s * RPT
        nk = ept // C
        pltpu.sync_copy(zeros_h.at[pl.ds(r0, RPT)], acc.at[pl.ds(r0, RPT)])
        pltpu.sync_copy(ones_h, ones_v)
        plsc.subcore_barrier()

        def load(kk, idx):
            base = pl.multiple_of(gwid * ept + kk * C, 8)
            pltpu.async_copy(dst_h.at[pl.ds(base, C)], idx, sem)

        def wait_load(kk, idx):
            base = pl.multiple_of(gwid * ept + kk * C, 8)
            pltpu.make_async_copy(dst_h.at[pl.ds(base, C)], idx, sem).wait()

        load(0, idx_a)

        def step(kk, cur, nxt):
            wait_load(kk, cur)

            @pl.when(kk + 1 < nk)
            def _():
                load(kk + 1, nxt)

            pltpu.sync_copy(ones_v, acc.at[cur], add=True)

        def body(kk, carry):
            @pl.when(kk % 2 == 0)
            def _():
                step(kk, idx_a, idx_b)

            @pl.when(kk % 2 == 1)
            def _():
                step(kk, idx_b, idx_a)

            return carry

        lax.fori_loop(0, nk, body, 0)
        plsc.subcore_barrier()
        pltpu.sync_copy(acc.at[pl.ds(r0, RPT)], out_h.at[c, pl.ds(r0, RPT)])

    return k(dst, ones_c, zeros16)


def _sc_aggregate(table, src, dst, zeros, width, chunk, tiled):
    ept = E // NW
    nk = ept // chunk

    @functools.partial(
        pl.kernel,
        out_type=jax.ShapeDtypeStruct((NC, N_PAD, width), jnp.float32),
        mesh=_mesh(),
        scratch_types=[
            pltpu.VMEM((chunk,), jnp.int32),
            pltpu.VMEM((chunk,), jnp.int32),
            pltpu.VMEM((chunk,), jnp.int32),
            pltpu.VMEM((chunk, width), jnp.float32),
            pltpu.VMEM((chunk, width), jnp.float32),
            pltpu.VMEM_SHARED((N_PAD, width), jnp.float32),
            pltpu.SemaphoreType.DMA,
        ],
        compiler_params=None if tiled else _SC_PARAMS,
    )
    def k(table_h, src_h, dst_h, zeros_h, out_h, src_a, src_b, dst_v,
          rows_a, rows_b, acc, sem_g):
        c = lax.axis_index("c")
        s = lax.axis_index("s")
        gwid = c * NS + s
        r0 = s * RPT
        pltpu.sync_copy(zeros_h.at[pl.ds(r0, RPT)], acc.at[pl.ds(r0, RPT)])
        plsc.subcore_barrier()

        def load_gather(kk, idx, rows):
            base = pl.multiple_of(gwid * ept + kk * chunk, 8)
            pltpu.sync_copy(src_h.at[pl.ds(base, chunk)], idx)
            pltpu.async_copy(table_h.at[idx], rows, sem_g)

        def wait_gather(idx, rows):
            pltpu.make_async_copy(table_h.at[idx], rows, sem_g).wait()

        def scatter(kk, rows):
            base = pl.multiple_of(gwid * ept + kk * chunk, 8)
            pltpu.sync_copy(dst_h.at[pl.ds(base, chunk)], dst_v)
            pltpu.sync_copy(rows, acc.at[dst_v], add=True)

        load_gather(0, src_a, rows_a)

        def step(kk, idx_c, rows_c, idx_n, rows_n):
            @pl.when(kk + 1 < nk)
            def _():
                load_gather(kk + 1, idx_n, rows_n)

            wait_gather(idx_c, rows_c)
            scatter(kk, rows_c)

        def body(kk, carry):
            @pl.when(kk % 2 == 0)
            def _():
                step(kk, src_a, rows_a, src_b, rows_b)

            @pl.when(kk % 2 == 1)
            def _():
                step(kk, src_b, rows_b, src_a, rows_a)

            return carry

        lax.fori_loop(0, nk, body, 0)
        plsc.subcore_barrier()
        pltpu.sync_copy(acc.at[pl.ds(r0, RPT)], out_h.at[c, pl.ds(r0, RPT)])

    return k(table, src, dst, zeros)


def _tc_layer1(xp, W1, deg8):
    BN = 1024
    grid = (N_PAD // BN,)

    def body(x_ref, w_ref, d_ref, t_ref, dinv_ref):
        d = d_ref[0] + d_ref[1]
        deg = jnp.sum(d, axis=1, keepdims=True) + 1.0
        dinv = jax.lax.rsqrt(deg)
        h = jnp.dot(x_ref[...], w_ref[...],
                    preferred_element_type=jnp.float32)
        t_ref[...] = h * jnp.broadcast_to(dinv, (BN, D_H))
        dinv_ref[...] = jnp.broadcast_to(dinv, (BN, DEG_W))

    return pl.pallas_call(
        body,
        grid=grid,
        in_specs=[
            pl.BlockSpec((BN, D_IN), lambda i: (i, 0)),
            pl.BlockSpec((D_IN, D_H), lambda i: (0, 0)),
            pl.BlockSpec((NC, BN, DEG_W), lambda i: (0, i, 0)),
        ],
        out_specs=[
            pl.BlockSpec((BN, D_H), lambda i: (i, 0)),
            pl.BlockSpec((BN, DEG_W), lambda i: (i, 0)),
        ],
        out_shape=[
            jax.ShapeDtypeStruct((N_PAD, D_H), jnp.float32),
            jax.ShapeDtypeStruct((N_PAD, DEG_W), jnp.float32),
        ],
    )(xp, W1, deg8)


def _tc_layer2(agg1, h1p, dinv16, b1, W2):
    BN = 1024
    grid = (N_PAD // BN,)

    def body(a_ref, h_ref, d_ref, b_ref, w_ref, out_ref):
        dinv = jnp.broadcast_to(d_ref[...][:, :1], (BN, D_H))
        full = a_ref[0] + a_ref[1] + h_ref[...]
        z = full * dinv + b_ref[...]
        z = jnp.maximum(z, 0.0)
        h2 = jnp.dot(z, w_ref[...], preferred_element_type=jnp.float32)
        out_ref[...] = h2 * dinv[:, :D_OUT]

    return pl.pallas_call(
        body,
        grid=grid,
        in_specs=[
            pl.BlockSpec((NC, BN, D_H), lambda i: (0, i, 0)),
            pl.BlockSpec((BN, D_H), lambda i: (i, 0)),
            pl.BlockSpec((BN, DEG_W), lambda i: (i, 0)),
            pl.BlockSpec((1, D_H), lambda i: (0, 0)),
            pl.BlockSpec((D_H, D_OUT), lambda i: (0, 0)),
        ],
        out_specs=pl.BlockSpec((BN, D_OUT), lambda i: (i, 0)),
        out_shape=jax.ShapeDtypeStruct((N_PAD, D_OUT), jnp.float32),
    )(agg1, h1p, dinv16, b1, W2)


def _tc_head(agg2, h2p, dinv16, b2):
    BN = 1024
    grid = (N_PAD // BN,)

    def body(a_ref, h_ref, d_ref, b_ref, out_ref):
        dinv = jnp.broadcast_to(d_ref[...][:, :1], (BN, D_OUT))
        y = (a_ref[0] + a_ref[1] + h_ref[...]) * dinv + b_ref[...]
        m = jnp.max(y, axis=1, keepdims=True)
        lse = jnp.log(jnp.sum(jnp.exp(y - m), axis=1, keepdims=True)) + m
        out_ref[...] = y - lse

    return pl.pallas_call(
        body,
        grid=grid,
        in_specs=[
            pl.BlockSpec((NC, BN, D_OUT), lambda i: (0, i, 0)),
            pl.BlockSpec((BN, D_OUT), lambda i: (i, 0)),
            pl.BlockSpec((BN, DEG_W), lambda i: (i, 0)),
            pl.BlockSpec((1, D_OUT), lambda i: (0, 0)),
        ],
        out_specs=pl.BlockSpec((BN, D_OUT), lambda i: (i, 0)),
        out_shape=jax.ShapeDtypeStruct((N_PAD, D_OUT), jnp.float32),
    )(agg2, h2p, dinv16, b2)


def kernel(x, edge_index, W1, b1, W2, b2):
    src = edge_index[0]
    dst = edge_index[1]
    xp = jnp.pad(x, ((0, N_PAD - N), (0, 0)))
    ones_c = jnp.full((C, DEG_W), 1.0 / DEG_W, jnp.float32)
    zeros8 = jnp.zeros((N_PAD, DEG_W), jnp.float32)
    zeros_w = jnp.zeros((N_PAD, D_H), jnp.float32)
    zeros_f = jnp.zeros((N_PAD, DF), jnp.float32)

    deg8 = _sc_degree(dst, ones_c, zeros8)
    h1p, dinv16 = _tc_layer1(xp, W1, deg8)
    agg1 = _sc_aggregate(h1p, src, dst, zeros_w, D_H, 80, True)
    h2p = _tc_layer2(agg1, h1p, dinv16, b1.reshape(1, D_H), W2)
    agg2 = _sc_aggregate(h2p, src, dst, zeros_f, DF, 200, False)
    out = _tc_head(agg2, h2p, dinv16, b2.reshape(1, D_OUT))
    return out[:N]


# speedup vs baseline: 1.1799x; 1.1799x over previous
import functools

import jax
import jax.numpy as jnp
from jax import lax
from jax.experimental import pallas as pl
from jax.experimental.pallas import tpu as pltpu
from jax.experimental.pallas import tpu_sc as plsc

N, E, D_IN, D_H, D_OUT = 10000, 320000, 128, 128, 64
NC, NS = 2, 16
NW = NC * NS
N_PAD = 10240
RPT = N_PAD // NS
C = 400
DEG_W = 16
DF = 64


def _mesh():
    return plsc.VectorSubcoreMesh(core_axis_name="c", subcore_axis_name="s")


_SC_PARAMS = pltpu.CompilerParams(use_tc_tiling_on_sc=False)


def _sc_degree(dst, ones_c, zeros16):
    ept = E // NW

    @functools.partial(
        pl.kernel,
        out_type=jax.ShapeDtypeStruct((NC, N_PAD, DEG_W), jnp.float32),
        mesh=_mesh(),
        scratch_types=[
            pltpu.VMEM((C,), jnp.int32),
            pltpu.VMEM((C,), jnp.int32),
            pltpu.VMEM((C, DEG_W), jnp.float32),
            pltpu.VMEM_SHARED((N_PAD, DEG_W), jnp.float32),
            pltpu.SemaphoreType.DMA,
        ],
        compiler_params=_SC_PARAMS,
    )
    def k(dst_h, ones_h, zeros_h, out_h, idx_a, idx_b, ones_v, acc, sem):
        c = lax.axis_index("c")
        s = lax.axis_index("s")
        gwid = c * NS + s
        r0 = s * RPT
        nk = ept // C
        pltpu.sync_copy(zeros_h.at[pl.ds(r0, RPT)], acc.at[pl.ds(r0, RPT)])
        pltpu.sync_copy(ones_h, ones_v)
        plsc.subcore_barrier()

        def load(kk, idx):
            base = pl.multiple_of(gwid * ept + kk * C, 8)
            pltpu.async_copy(dst_h.at[pl.ds(base, C)], idx, sem)

        def wait_load(kk, idx):
            base = pl.multiple_of(gwid * ept + kk * C, 8)
            pltpu.make_async_copy(dst_h.at[pl.ds(base, C)], idx, sem).wait()

        load(0, idx_a)

        def step(kk, cur, nxt):
            wait_load(kk, cur)

            @pl.when(kk + 1 < nk)
            def _():
                load(kk + 1, nxt)

            pltpu.sync_copy(ones_v, acc.at[cur], add=True)

        def body(kk, carry):
            @pl.when(kk % 2 == 0)
            def _():
                step(kk, idx_a, idx_b)

            @pl.when(kk % 2 == 1)
            def _():
                step(kk, idx_b, idx_a)

            return carry

        lax.fori_loop(0, nk, body, 0)
        plsc.subcore_barrier()
        pltpu.sync_copy(acc.at[pl.ds(r0, RPT)], out_h.at[c, pl.ds(r0, RPT)])

    return k(dst, ones_c, zeros16)


def _sc_aggregate(table, src, dst, zeros, feature_split):
    ept = (E // NS) if feature_split else (E // NW)
    nk = ept // C

    @functools.partial(
        pl.kernel,
        out_type=jax.ShapeDtypeStruct((NC, N_PAD, DF), jnp.float32),
        mesh=_mesh(),
        scratch_types=[
            pltpu.VMEM((C,), jnp.int32),
            pltpu.VMEM((C,), jnp.int32),
            pltpu.VMEM((C,), jnp.int32),
            pltpu.VMEM((C,), jnp.int32),
            pltpu.VMEM((C, DF), jnp.float32),
            pltpu.VMEM((C, DF), jnp.float32),
            pltpu.VMEM_SHARED((N_PAD, DF), jnp.float32),
            pltpu.SemaphoreType.DMA,
            pltpu.SemaphoreType.DMA,
        ],
        compiler_params=_SC_PARAMS,
    )
    def k(table_h, src_h, dst_h, zeros_h, out_h, src_a, src_b, dst_a,
          dst_b, rows_a, rows_b, acc, sem_g, sem_s):
        c = lax.axis_index("c")
        s = lax.axis_index("s")
        gwid = s if feature_split else c * NS + s
        r0 = s * RPT
        pltpu.sync_copy(zeros_h.at[pl.ds(r0, RPT)], acc.at[pl.ds(r0, RPT)])
        plsc.subcore_barrier()

        def load_gather(kk, idx, rows):
            base = pl.multiple_of(gwid * ept + kk * C, 8)
            pltpu.sync_copy(src_h.at[pl.ds(base, C)], idx)
            if feature_split:
                off = jnp.broadcast_to(c * N_PAD, (16,)).astype(jnp.int32)
                for j in range(C // 16):
                    sl = pl.ds(j * 16, 16)
                    idx[sl] = idx[sl] + off
            pltpu.async_copy(table_h.at[idx], rows, sem_g)

        def wait_gather(idx, rows):
            pltpu.make_async_copy(table_h.at[idx], rows, sem_g).wait()

        def scatter(kk, dstb, rows):
            base = pl.multiple_of(gwid * ept + kk * C, 8)
            pltpu.sync_copy(dst_h.at[pl.ds(base, C)], dstb)
            pltpu.async_copy(rows, acc.at[dstb], sem_s, add=True)

        def wait_scatter(dstb, rows):
            pltpu.make_async_copy(rows, acc.at[dstb], sem_s).wait()

        load_gather(0, src_a, rows_a)

        def step(kk, idx_c, dst_c, rows_c, idx_n, dst_n, rows_n):
            @pl.when(kk >= 1)
            def _():
                wait_scatter(dst_n, rows_n)

            @pl.when(kk + 1 < nk)
            def _():
                load_gather(kk + 1, idx_n, rows_n)

            wait_gather(idx_c, rows_c)
            scatter(kk, dst_c, rows_c)

        def body(kk, carry):
            @pl.when(kk % 2 == 0)
            def _():
                step(kk, src_a, dst_a, rows_a, src_b, dst_b, rows_b)

            @pl.when(kk % 2 == 1)
            def _():
                step(kk, src_b, dst_b, rows_b, src_a, dst_a, rows_a)

            return carry

        lax.fori_loop(0, nk, body, 0)
        if (nk - 1) % 2 == 0:
            wait_scatter(dst_a, rows_a)
        else:
            wait_scatter(dst_b, rows_b)
        plsc.subcore_barrier()
        pltpu.sync_copy(acc.at[pl.ds(r0, RPT)], out_h.at[c, pl.ds(r0, RPT)])

    return k(table, src, dst, zeros)


def _tc_layer1(xp, W1, deg16):
    BN = 1024
    nb = N_PAD // BN
    grid = (nb, 2)

    def body(x_ref, w_ref, d_ref, t_ref, dinv_ref):
        d = d_ref[0] + d_ref[1]
        deg = jnp.sum(d, axis=1, keepdims=True) + 1.0
        dinv = jax.lax.rsqrt(deg)
        h = jnp.dot(x_ref[...], w_ref[0],
                    preferred_element_type=jnp.float32)
        t_ref[0] = h * jnp.broadcast_to(dinv, (BN, DF))
        dinv_ref[...] = jnp.broadcast_to(dinv, (BN, DEG_W))

    return pl.pallas_call(
        body,
        grid=grid,
        in_specs=[
            pl.BlockSpec((BN, D_IN), lambda i, j: (i, 0)),
            pl.BlockSpec((1, D_IN, DF), lambda i, j: (j, 0, 0)),
            pl.BlockSpec((NC, BN, DEG_W), lambda i, j: (0, i, 0)),
        ],
        out_specs=[
            pl.BlockSpec((1, BN, DF), lambda i, j: (j, i, 0)),
            pl.BlockSpec((BN, DEG_W), lambda i, j: (i, 0)),
        ],
        out_shape=[
            jax.ShapeDtypeStruct((2, N_PAD, DF), jnp.float32),
            jax.ShapeDtypeStruct((N_PAD, DEG_W), jnp.float32),
        ],
    )(xp, W1, deg16)


def _tc_layer2(agg1, h1p, dinv16, b1, W2):
    BN = 1024
    grid = (N_PAD // BN,)

    def body(a_ref, h_ref, d_ref, b_ref, w_ref, out_ref):
        dinv = jnp.broadcast_to(d_ref[...][:, :1], (BN, D_H))
        full = jnp.concatenate([a_ref[0] + h_ref[0], a_ref[1] + h_ref[1]],
                               axis=1)
        z = full * dinv + b_ref[...]
        z = jnp.maximum(z, 0.0)
        h2 = jnp.dot(z, w_ref[...], preferred_element_type=jnp.float32)
        out_ref[...] = h2 * dinv[:, :D_OUT]

    return pl.pallas_call(
        body,
        grid=grid,
        in_specs=[
            pl.BlockSpec((NC, BN, DF), lambda i: (0, i, 0)),
            pl.BlockSpec((NC, BN, DF), lambda i: (0, i, 0)),
            pl.BlockSpec((BN, DEG_W), lambda i: (i, 0)),
            pl.BlockSpec((1, D_H), lambda i: (0, 0)),
            pl.BlockSpec((D_H, D_OUT), lambda i: (0, 0)),
        ],
        out_specs=pl.BlockSpec((BN, D_OUT), lambda i: (i, 0)),
        out_shape=jax.ShapeDtypeStruct((N_PAD, D_OUT), jnp.float32),
    )(agg1, h1p, dinv16, b1, W2)


def _tc_head(agg2, h2p, dinv16, b2):
    BN = 1024
    grid = (N_PAD // BN,)

    def body(a_ref, h_ref, d_ref, b_ref, out_ref):
        dinv = jnp.broadcast_to(d_ref[...][:, :1], (BN, D_OUT))
        y = (a_ref[0] + a_ref[1] + h_ref[...]) * dinv + b_ref[...]
        m = jnp.max(y, axis=1, keepdims=True)
        lse = jnp.log(jnp.sum(jnp.exp(y - m), axis=1, keepdims=True)) + m
        out_ref[...] = y - lse

    return pl.pallas_call(
        body,
        grid=grid,
        in_specs=[
            pl.BlockSpec((NC, BN, D_OUT), lambda i: (0, i, 0)),
            pl.BlockSpec((BN, D_OUT), lambda i: (i, 0)),
            pl.BlockSpec((BN, DEG_W), lambda i: (i, 0)),
            pl.BlockSpec((1, D_OUT), lambda i: (0, 0)),
        ],
        out_specs=pl.BlockSpec((BN, D_OUT), lambda i: (i, 0)),
        out_shape=jax.ShapeDtypeStruct((N_PAD, D_OUT), jnp.float32),
    )(agg2, h2p, dinv16, b2)


def kernel(x, edge_index, W1, b1, W2, b2):
    src = edge_index[0]
    dst = edge_index[1]
    xp = jnp.pad(x, ((0, N_PAD - N), (0, 0)))
    ones_c = jnp.full((C, DEG_W), 1.0 / DEG_W, jnp.float32)
    zeros16 = jnp.zeros((N_PAD, DEG_W), jnp.float32)
    zeros_f = jnp.zeros((N_PAD, DF), jnp.float32)

    W1h = W1.reshape(D_IN, 2, DF).transpose(1, 0, 2)
    deg16 = _sc_degree(dst, ones_c, zeros16)
    h1p, dinv16 = _tc_layer1(xp, W1h, deg16)
    table1 = h1p.reshape(2 * N_PAD, DF)
    agg1 = _sc_aggregate(table1, src, dst, zeros_f, True)
    h2p = _tc_layer2(agg1, h1p, dinv16, b1.reshape(1, D_H), W2)
    agg2 = _sc_aggregate(h2p, src, dst, zeros_f, False)
    out = _tc_head(agg2, h2p, dinv16, b2.reshape(1, D_OUT))
    return out[:N]


# speedup vs baseline: 1.2181x; 1.0324x over previous
import functools

import jax
import jax.numpy as jnp
from jax import lax
from jax.experimental import pallas as pl
from jax.experimental.pallas import tpu as pltpu
from jax.experimental.pallas import tpu_sc as plsc

N, E, D_IN, D_H, D_OUT = 10000, 320000, 128, 128, 64
NC, NS = 2, 16
NW = NC * NS
N_PAD = 10240
RPT = N_PAD // NS
C = 400
DEG_W = 16
DF = 64


def _mesh():
    return plsc.VectorSubcoreMesh(core_axis_name="c", subcore_axis_name="s")


_SC_PARAMS = pltpu.CompilerParams(use_tc_tiling_on_sc=False)


def _sc_degree(dst, ones_c, zeros16):
    ept = E // NW

    @functools.partial(
        pl.kernel,
        out_type=jax.ShapeDtypeStruct((NC, N_PAD, DEG_W), jnp.float32),
        mesh=_mesh(),
        scratch_types=[
            pltpu.VMEM((C,), jnp.int32),
            pltpu.VMEM((C,), jnp.int32),
            pltpu.VMEM((C, DEG_W), jnp.float32),
            pltpu.VMEM_SHARED((N_PAD, DEG_W), jnp.float32),
            pltpu.SemaphoreType.DMA,
        ],
        compiler_params=_SC_PARAMS,
    )
    def k(dst_h, ones_h, zeros_h, out_h, idx_a, idx_b, ones_v, acc, sem):
        c = lax.axis_index("c")
        s = lax.axis_index("s")
        gwid = c * NS + s
        r0 = s * RPT
        nk = ept // C
        pltpu.sync_copy(zeros_h.at[pl.ds(r0, RPT)], acc.at[pl.ds(r0, RPT)])
        pltpu.sync_copy(ones_h, ones_v)
        plsc.subcore_barrier()

        def load(kk, idx):
            base = pl.multiple_of(gwid * ept + kk * C, 8)
            pltpu.async_copy(dst_h.at[pl.ds(base, C)], idx, sem)

        def wait_load(kk, idx):
            base = pl.multiple_of(gwid * ept + kk * C, 8)
            pltpu.make_async_copy(dst_h.at[pl.ds(base, C)], idx, sem).wait()

        load(0, idx_a)

        def step(kk, cur, nxt):
            wait_load(kk, cur)

            @pl.when(kk + 1 < nk)
            def _():
                load(kk + 1, nxt)

            pltpu.sync_copy(ones_v, acc.at[cur], add=True)

        def body(kk, carry):
            @pl.when(kk % 2 == 0)
            def _():
                step(kk, idx_a, idx_b)

            @pl.when(kk % 2 == 1)
            def _():
                step(kk, idx_b, idx_a)

            return carry

        lax.fori_loop(0, nk, body, 0)
        plsc.subcore_barrier()
        pltpu.sync_copy(acc.at[pl.ds(r0, RPT)], out_h.at[c, pl.ds(r0, RPT)])

    return k(dst, ones_c, zeros16)


def _sc_aggregate(table, src, dst, zeros, feature_split):
    ept = (E // NS) if feature_split else (E // NW)
    nk = ept // C

    @functools.partial(
        pl.kernel,
        out_type=jax.ShapeDtypeStruct((NC, N_PAD, DF), jnp.float32),
        mesh=_mesh(),
        scratch_types=[
            pltpu.VMEM((C,), jnp.int32),
            pltpu.VMEM((C,), jnp.int32),
            pltpu.VMEM((C,), jnp.int32),
            pltpu.VMEM((C,), jnp.int32),
            pltpu.VMEM((C, DF), jnp.float32),
            pltpu.VMEM((C, DF), jnp.float32),
            pltpu.VMEM_SHARED((N_PAD, DF), jnp.float32),
            pltpu.SemaphoreType.DMA,
            pltpu.SemaphoreType.DMA,
        ],
        compiler_params=_SC_PARAMS,
    )
    def k(table_h, src_h, dst_h, zeros_h, out_h, src_a, src_b, dst_a,
          dst_b, rows_a, rows_b, acc, sem_g, sem_s):
        c = lax.axis_index("c")
        s = lax.axis_index("s")
        gwid = s if feature_split else c * NS + s
        r0 = s * RPT
        pltpu.sync_copy(zeros_h.at[pl.ds(r0, RPT)], acc.at[pl.ds(r0, RPT)])
        plsc.subcore_barrier()

        def load_gather(kk, idx, rows):
            base = pl.multiple_of(gwid * ept + kk * C, 8)
            pltpu.sync_copy(src_h.at[pl.ds(base, C)], idx)
            if feature_split:
                off = jnp.broadcast_to(c * N, (16,)).astype(jnp.int32)
                for j in range(C // 16):
                    sl = pl.ds(j * 16, 16)
                    idx[sl] = idx[sl] + off
            pltpu.async_copy(table_h.at[idx], rows, sem_g)

        def wait_gather(idx, rows):
            pltpu.make_async_copy(table_h.at[idx], rows, sem_g).wait()

        def scatter(kk, dstb, rows):
            base = pl.multiple_of(gwid * ept + kk * C, 8)
            pltpu.sync_copy(dst_h.at[pl.ds(base, C)], dstb)
            pltpu.async_copy(rows, acc.at[dstb], sem_s, add=True)

        def wait_scatter(dstb, rows):
            pltpu.make_async_copy(rows, acc.at[dstb], sem_s).wait()

        load_gather(0, src_a, rows_a)

        def step(kk, idx_c, dst_c, rows_c, idx_n, dst_n, rows_n):
            @pl.when(kk >= 1)
            def _():
                wait_scatter(dst_n, rows_n)

            @pl.when(kk + 1 < nk)
            def _():
                load_gather(kk + 1, idx_n, rows_n)

            wait_gather(idx_c, rows_c)
            scatter(kk, dst_c, rows_c)

        def body(kk, carry):
            @pl.when(kk % 2 == 0)
            def _():
                step(kk, src_a, dst_a, rows_a, src_b, dst_b, rows_b)

            @pl.when(kk % 2 == 1)
            def _():
                step(kk, src_b, dst_b, rows_b, src_a, dst_a, rows_a)

            return carry

        lax.fori_loop(0, nk, body, 0)
        if (nk - 1) % 2 == 0:
            wait_scatter(dst_a, rows_a)
        else:
            wait_scatter(dst_b, rows_b)
        plsc.subcore_barrier()
        pltpu.sync_copy(acc.at[pl.ds(r0, RPT)], out_h.at[c, pl.ds(r0, RPT)])

    return k(table, src, dst, zeros)


BN = 1000


def _tc_mm(x, W1):
    grid = (N // BN,)

    def body(x_ref, w_ref, h_ref):
        h_ref[...] = jnp.dot(x_ref[...], w_ref[...],
                             preferred_element_type=jnp.float32)

    return pl.pallas_call(
        body,
        grid=grid,
        in_specs=[
            pl.BlockSpec((BN, D_IN), lambda i: (i, 0)),
            pl.BlockSpec((D_IN, D_H), lambda i: (0, 0)),
        ],
        out_specs=pl.BlockSpec((BN, D_H), lambda i: (i, 0)),
        out_shape=jax.ShapeDtypeStruct((N, D_H), jnp.float32),
    )(x, W1)


def _tc_scale(h1, deg16):
    grid = (N // BN,)

    def body(h_ref, d_ref, t_ref, dinv_ref):
        d = d_ref[0] + d_ref[1]
        deg = jnp.sum(d, axis=1, keepdims=True) + 1.0
        dinv = jax.lax.rsqrt(deg)
        hs = h_ref[...] * jnp.broadcast_to(dinv, (BN, D_H))
        t_ref[0] = hs[:, :DF]
        t_ref[1] = hs[:, DF:]
        dinv_ref[...] = jnp.broadcast_to(dinv, (BN, DEG_W))

    return pl.pallas_call(
        body,
        grid=grid,
        in_specs=[
            pl.BlockSpec((BN, D_H), lambda i: (i, 0)),
            pl.BlockSpec((NC, BN, DEG_W), lambda i: (0, i, 0)),
        ],
        out_specs=[
            pl.BlockSpec((2, BN, DF), lambda i: (0, i, 0)),
            pl.BlockSpec((BN, DEG_W), lambda i: (i, 0)),
        ],
        out_shape=[
            jax.ShapeDtypeStruct((2, N, DF), jnp.float32),
            jax.ShapeDtypeStruct((N, DEG_W), jnp.float32),
        ],
    )(h1, deg16)


def _tc_layer2(agg1, h1p, dinv16, b1, W2):
    grid = (N // BN,)

    def body(a_ref, h_ref, d_ref, b_ref, w_ref, out_ref):
        dinv = jnp.broadcast_to(d_ref[...][:, :1], (BN, D_H))
        full = jnp.concatenate([a_ref[0] + h_ref[0], a_ref[1] + h_ref[1]],
                               axis=1)
        z = full * dinv + b_ref[...]
        z = jnp.maximum(z, 0.0)
        h2 = jnp.dot(z, w_ref[...], preferred_element_type=jnp.float32)
        out_ref[...] = h2 * dinv[:, :D_OUT]

    return pl.pallas_call(
        body,
        grid=grid,
        in_specs=[
            pl.BlockSpec((NC, BN, DF), lambda i: (0, i, 0)),
            pl.BlockSpec((NC, BN, DF), lambda i: (0, i, 0)),
            pl.BlockSpec((BN, DEG_W), lambda i: (i, 0)),
            pl.BlockSpec((1, D_H), lambda i: (0, 0)),
            pl.BlockSpec((D_H, D_OUT), lambda i: (0, 0)),
        ],
        out_specs=pl.BlockSpec((BN, D_OUT), lambda i: (i, 0)),
        out_shape=jax.ShapeDtypeStruct((N, D_OUT), jnp.float32),
    )(agg1, h1p, dinv16, b1, W2)


def _tc_head(agg2, h2p, dinv16, b2):
    grid = (N // BN,)

    def body(a_ref, h_ref, d_ref, b_ref, out_ref):
        dinv = jnp.broadcast_to(d_ref[...][:, :1], (BN, D_OUT))
        y = (a_ref[0] + a_ref[1] + h_ref[...]) * dinv + b_ref[...]
        m = jnp.max(y, axis=1, keepdims=True)
        lse = jnp.log(jnp.sum(jnp.exp(y - m), axis=1, keepdims=True)) + m
        out_ref[...] = y - lse

    return pl.pallas_call(
        body,
        grid=grid,
        in_specs=[
            pl.BlockSpec((NC, BN, D_OUT), lambda i: (0, i, 0)),
            pl.BlockSpec((BN, D_OUT), lambda i: (i, 0)),
            pl.BlockSpec((BN, DEG_W), lambda i: (i, 0)),
            pl.BlockSpec((1, D_OUT), lambda i: (0, 0)),
        ],
        out_specs=pl.BlockSpec((BN, D_OUT), lambda i: (i, 0)),
        out_shape=jax.ShapeDtypeStruct((N, D_OUT), jnp.float32),
    )(agg2, h2p, dinv16, b2)


def kernel(x, edge_index, W1, b1, W2, b2):
    src = edge_index[0]
    dst = edge_index[1]
    ones_c = jnp.full((C, DEG_W), 1.0 / DEG_W, jnp.float32)
    zeros16 = jnp.zeros((N_PAD, DEG_W), jnp.float32)
    zeros_f = jnp.zeros((N_PAD, DF), jnp.float32)

    h1 = _tc_mm(x, W1)
    deg16 = _sc_degree(dst, ones_c, zeros16)
    h1p, dinv16 = _tc_scale(h1, deg16)
    table1 = h1p.reshape(2 * N, DF)
    agg1 = _sc_aggregate(table1, src, dst, zeros_f, True)
    h2p = _tc_layer2(agg1, h1p, dinv16, b1.reshape(1, D_H), W2)
    agg2 = _sc_aggregate(h2p, src, dst, zeros_f, False)
    return _tc_head(agg2, h2p, dinv16, b2.reshape(1, D_OUT))
